# Initial kernel scaffold; baseline (speedup 1.0000x reference)
import functools

import jax
import jax.numpy as jnp
from jax import lax
from jax.experimental import pallas as pl
from jax.experimental.pallas import tpu as pltpu
from jax.experimental.pallas import tpu_sc as plsc

N = 10000
E = 320000
D = 128

NC = 2
NS = 16
NT = NC * NS

CHUNK = 128
NCHUNKS = E // CHUNK
CPT = (NCHUNKS + NT - 1) // NT
ROWS_PER_TILE = N // NS


def _sc_agg_body(with_cnt, x_hbm, src_hbm, dst_hbm, *refs):
    if with_cnt:
        (agg0_hbm, agg1_hbm, cnt0_hbm, cnt1_hbm,
         agg_sh, rows_v, isrc_v, idst_v, cnt_sh, ones_v, zc_v) = refs
    else:
        (agg0_hbm, agg1_hbm, agg_sh, rows_v, isrc_v, idst_v) = refs

    cid = lax.axis_index("c")
    sid = lax.axis_index("s")
    wid = cid * NS + sid

    @pl.loop(0, CHUNK)
    def _(r):
        @pl.loop(0, D, step=16)
        def _(j):
            rows_v[r, pl.ds(j, 16)] = jnp.zeros((16,), jnp.float32)

    r0 = sid * ROWS_PER_TILE
    @pl.loop(0, 4)
    def _(k):
        pltpu.sync_copy(rows_v, agg_sh.at[pl.ds(r0 + k * CHUNK, CHUNK)])
    pltpu.sync_copy(rows_v.at[pl.ds(0, 113)], agg_sh.at[pl.ds(r0 + 512, 113)])

    if with_cnt:
        @pl.loop(0, CHUNK, step=16)
        def _(j):
            ones_v[pl.ds(j, 16)] = jnp.ones((16,), jnp.float32)

        @pl.when(sid == 0)
        def _():
            @pl.loop(0, N, step=16)
            def _(j):
                zc_v[pl.ds(j, 16)] = jnp.zeros((16,), jnp.float32)
            pltpu.sync_copy(zc_v, cnt_sh)

    plsc.subcore_barrier()

    @pl.loop(0, CPT)
    def _(i):
        c_g = i * NT + wid

        @pl.when(c_g < NCHUNKS)
        def _():
            base = c_g * CHUNK
            pltpu.sync_copy(src_hbm.at[pl.ds(base, CHUNK)], isrc_v)
            pltpu.sync_copy(dst_hbm.at[pl.ds(base, CHUNK)], idst_v.at[0])
            pltpu.sync_copy(x_hbm.at[isrc_v], rows_v)
            pltpu.sync_copy(rows_v, agg_sh.at[idst_v.at[0]], add=True)
            if with_cnt:
                pltpu.sync_copy(ones_v, cnt_sh.at[idst_v.at[0]], add=True)

    plsc.subcore_barrier()

    @pl.when(cid == 0)
    def _():
        pltpu.sync_copy(agg_sh.at[pl.ds(r0, ROWS_PER_TILE)],
                        agg0_hbm.at[pl.ds(r0, ROWS_PER_TILE)])

    @pl.when(cid == 1)
    def _():
        pltpu.sync_copy(agg_sh.at[pl.ds(r0, ROWS_PER_TILE)],
                        agg1_hbm.at[pl.ds(r0, ROWS_PER_TILE)])

    if with_cnt:
        @pl.when((sid == 0) & (cid == 0))
        def _():
            pltpu.sync_copy(cnt_sh, cnt0_hbm)

        @pl.when((sid == 0) & (cid == 1))
        def _():
            pltpu.sync_copy(cnt_sh, cnt1_hbm)


def _make_sc_agg(with_cnt):
    mesh = plsc.VectorSubcoreMesh(core_axis_name="c", subcore_axis_name="s",
                                  num_cores=NC, num_subcores=NS)
    out_type = [jax.ShapeDtypeStruct((N, D), jnp.float32),
                jax.ShapeDtypeStruct((N, D), jnp.float32)]
    scratch = [
        pltpu.VMEM_SHARED((N, D), jnp.float32),
        pltpu.VMEM((CHUNK, D), jnp.float32),
        pltpu.VMEM((CHUNK,), jnp.int32),
        pltpu.VMEM((1, CHUNK), jnp.int32),
    ]
    if with_cnt:
        out_type += [jax.ShapeDtypeStruct((N,), jnp.float32),
                     jax.ShapeDtypeStruct((N,), jnp.float32)]
        scratch += [
            pltpu.VMEM_SHARED((N,), jnp.float32),
            pltpu.VMEM((CHUNK,), jnp.float32),
            pltpu.VMEM((N,), jnp.float32),
        ]
    return pl.kernel(functools.partial(_sc_agg_body, with_cnt),
                     out_type=tuple(out_type), mesh=mesh,
                     scratch_types=scratch)


_sc_agg_cnt = _make_sc_agg(True)
_sc_agg = _make_sc_agg(False)


_BQ = 400
_DN = (((1,), (1,)), ((), ()))


def _dense_body(relu, a0_ref, a1_ref, c0_ref, c1_ref, x_ref, wl_ref, bl_ref,
                wr_ref, o_ref):
    cnt = c0_ref[...] + c1_ref[...]
    inv = 1.0 / jnp.maximum(cnt, 1.0)
    mean = (a0_ref[...] + a1_ref[...]) * inv
    acc = lax.dot_general(mean, wl_ref[...], _DN,
                          preferred_element_type=jnp.float32,
                          precision=lax.Precision.HIGHEST)
    acc = acc + lax.dot_general(x_ref[...], wr_ref[...], _DN,
                                preferred_element_type=jnp.float32,
                                precision=lax.Precision.HIGHEST)
    acc = acc + bl_ref[...]
    if relu:
        acc = jnp.maximum(acc, 0.0)
    o_ref[...] = acc


def _dense(relu, a0, a1, c0, c1, x, wl, bl, wr):
    row_spec = pl.BlockSpec((_BQ, D), lambda i: (i, 0))
    cnt_spec = pl.BlockSpec((_BQ, 1), lambda i: (i, 0))
    w_spec = pl.BlockSpec((D, D), lambda i: (0, 0))
    b_spec = pl.BlockSpec((1, D), lambda i: (0, 0))
    return pl.pallas_call(
        functools.partial(_dense_body, relu),
        grid=(N // _BQ,),
        in_specs=[row_spec, row_spec, cnt_spec, cnt_spec, row_spec,
                  w_spec, b_spec, w_spec],
        out_specs=row_spec,
        out_shape=jax.ShapeDtypeStruct((N, D), jnp.float32),
    )(a0, a1, c0, c1, x, wl, bl, wr)


def kernel(x, edge_index, W1l, b1l, W1r, W2l, b2l, W2r):
    src = edge_index[0]
    dst = edge_index[1]
    agg0, agg1, cnt0, cnt1 = _sc_agg_cnt(x, src, dst)
    c0 = cnt0.reshape(N, 1)
    c1 = cnt1.reshape(N, 1)
    h = _dense(True, agg0, agg1, c0, c1, x, W1l, b1l.reshape(1, D), W1r)
    b0, b1_ = _sc_agg(h, src, dst)
    out = _dense(False, b0, b1_, c0, c1, h, W2l, b2l.reshape(1, D), W2r)
    return out


# speedup vs baseline: 6.5206x; 6.5206x over previous
import functools

import jax
import jax.numpy as jnp
from jax import lax
from jax.experimental import pallas as pl
from jax.experimental.pallas import tpu as pltpu
from jax.experimental.pallas import tpu_sc as plsc

N = 10000
E = 320000
D = 128

NC = 2
NS = 16
NT = NC * NS

CHUNK = 128
NCHUNKS = E // CHUNK
CPT = (NCHUNKS + NT - 1) // NT
ROW_STRIDE = 624


def _sc_agg_body(with_cnt, x_hbm, src_hbm, dst_hbm, *refs):
    if with_cnt:
        (agg0_hbm, agg1_hbm, cnt0_hbm, cnt1_hbm,
         agg_sh, rows_v, isrc_v, idst_v, cnt_sh, ones_v, zc_v) = refs
    else:
        (agg0_hbm, agg1_hbm, agg_sh, rows_v, isrc_v, idst_v) = refs

    cid = lax.axis_index("c")
    sid = lax.axis_index("s")
    wid = cid * NS + sid

    @pl.loop(0, CHUNK)
    def _(r):
        @pl.loop(0, D, step=16)
        def _(j):
            rows_v[r, pl.ds(j, 16)] = jnp.zeros((16,), jnp.float32)

    start = sid * ROW_STRIDE

    @pl.loop(0, 5)
    def _(k):
        pltpu.sync_copy(rows_v, agg_sh.at[pl.ds(start + k * CHUNK, CHUNK)])

    if with_cnt:
        @pl.loop(0, CHUNK, step=16)
        def _(j):
            ones_v[pl.ds(j, 16)] = jnp.ones((16,), jnp.float32)

        @pl.when(sid == 0)
        def _():
            @pl.loop(0, N, step=16)
            def _(j):
                zc_v[pl.ds(j, 16)] = jnp.zeros((16,), jnp.float32)
            pltpu.sync_copy(zc_v, cnt_sh)

    plsc.subcore_barrier()

    @pl.loop(0, CPT)
    def _(i):
        c_g = i * NT + wid

        @pl.when(c_g < NCHUNKS)
        def _():
            base = c_g * CHUNK
            pltpu.sync_copy(src_hbm.at[pl.ds(base, CHUNK)], isrc_v)
            pltpu.sync_copy(dst_hbm.at[pl.ds(base, CHUNK)], idst_v.at[0])
            pltpu.sync_copy(x_hbm.at[isrc_v], rows_v)
            pltpu.sync_copy(rows_v, agg_sh.at[idst_v.at[0]], add=True)
            if with_cnt:
                pltpu.sync_copy(ones_v, cnt_sh.at[idst_v.at[0]], add=True)

    plsc.subcore_barrier()

    def _copy_out(dst_hbm_ref):
        pltpu.sync_copy(agg_sh.at[pl.ds(start, 512)],
                        dst_hbm_ref.at[pl.ds(start, 512)])

        @pl.when(sid < NS - 1)
        def _():
            pltpu.sync_copy(agg_sh.at[pl.ds(start + 512, 112)],
                            dst_hbm_ref.at[pl.ds(start + 512, 112)])

        @pl.when(sid == NS - 1)
        def _():
            pltpu.sync_copy(agg_sh.at[pl.ds(start + 512, 128)],
                            dst_hbm_ref.at[pl.ds(start + 512, 128)])

    @pl.when(cid == 0)
    def _():
        _copy_out(agg0_hbm)

    @pl.when(cid == 1)
    def _():
        _copy_out(agg1_hbm)

    if with_cnt:
        @pl.when((sid == 0) & (cid == 0))
        def _():
            pltpu.sync_copy(cnt_sh, cnt0_hbm)

        @pl.when((sid == 0) & (cid == 1))
        def _():
            pltpu.sync_copy(cnt_sh, cnt1_hbm)


def _make_sc_agg(with_cnt):
    mesh = plsc.VectorSubcoreMesh(core_axis_name="c", subcore_axis_name="s",
                                  num_cores=NC, num_subcores=NS)
    out_type = [jax.ShapeDtypeStruct((N, D), jnp.float32),
                jax.ShapeDtypeStruct((N, D), jnp.float32)]
    scratch = [
        pltpu.VMEM_SHARED((N, D), jnp.float32),
        pltpu.VMEM((CHUNK, D), jnp.float32),
        pltpu.VMEM((CHUNK,), jnp.int32),
        pltpu.VMEM((1, CHUNK), jnp.int32),
    ]
    if with_cnt:
        out_type += [jax.ShapeDtypeStruct((N,), jnp.float32),
                     jax.ShapeDtypeStruct((N,), jnp.float32)]
        scratch += [
            pltpu.VMEM_SHARED((N,), jnp.float32),
            pltpu.VMEM((CHUNK,), jnp.float32),
            pltpu.VMEM((N,), jnp.float32),
        ]
    return pl.kernel(functools.partial(_sc_agg_body, with_cnt),
                     out_type=tuple(out_type), mesh=mesh,
                     scratch_types=scratch)


_sc_agg_cnt = _make_sc_agg(True)
_sc_agg = _make_sc_agg(False)


_BQ = 400
_DN = (((1,), (1,)), ((), ()))


def _dense_body(relu, a0_ref, a1_ref, c0_ref, c1_ref, x_ref, wl_ref, bl_ref,
                wr_ref, o_ref):
    cnt = c0_ref[...] + c1_ref[...]
    inv = 1.0 / jnp.maximum(cnt, 1.0)
    mean = (a0_ref[...] + a1_ref[...]) * inv
    acc = lax.dot_general(mean, wl_ref[...], _DN,
                          preferred_element_type=jnp.float32,
                          precision=lax.Precision.HIGHEST)
    acc = acc + lax.dot_general(x_ref[...], wr_ref[...], _DN,
                                preferred_element_type=jnp.float32,
                                precision=lax.Precision.HIGHEST)
    acc = acc + bl_ref[...]
    if relu:
        acc = jnp.maximum(acc, 0.0)
    o_ref[...] = acc


def _dense(relu, a0, a1, c0, c1, x, wl, bl, wr):
    row_spec = pl.BlockSpec((_BQ, D), lambda i: (i, 0))
    cnt_spec = pl.BlockSpec((_BQ, 1), lambda i: (i, 0))
    w_spec = pl.BlockSpec((D, D), lambda i: (0, 0))
    b_spec = pl.BlockSpec((1, D), lambda i: (0, 0))
    return pl.pallas_call(
        functools.partial(_dense_body, relu),
        grid=(N // _BQ,),
        in_specs=[row_spec, row_spec, cnt_spec, cnt_spec, row_spec,
                  w_spec, b_spec, w_spec],
        out_specs=row_spec,
        out_shape=jax.ShapeDtypeStruct((N, D), jnp.float32),
    )(a0, a1, c0, c1, x, wl, bl, wr)


def kernel(x, edge_index, W1l, b1l, W1r, W2l, b2l, W2r):
    src = edge_index[0]
    dst = edge_index[1]
    agg0, agg1, cnt0, cnt1 = _sc_agg_cnt(x, src, dst)
    c0 = cnt0.reshape(N, 1)
    c1 = cnt1.reshape(N, 1)
    h = _dense(True, agg0, agg1, c0, c1, x, W1l, b1l.reshape(1, D), W1r)
    b0, b1_ = _sc_agg(h, src, dst)
    out = _dense(False, b0, b1_, c0, c1, h, W2l, b2l.reshape(1, D), W2r)
    return out


# speedup vs baseline: 10.9063x; 1.6726x over previous
import functools

import jax
import jax.numpy as jnp
from jax import lax
from jax.experimental import pallas as pl
from jax.experimental.pallas import tpu as pltpu
from jax.experimental.pallas import tpu_sc as plsc

N = 10000
E = 320000
D = 128

NC = 2
NS = 16
NT = NC * NS

CHUNK = 128
NCHUNKS = E // CHUNK
CPT = (NCHUNKS + NT - 1) // NT
ROW_STRIDE = 624


def _sc_agg_body(with_cnt, x_hbm, src_hbm, dst_hbm, *refs):
    if with_cnt:
        (agg0_hbm, agg1_hbm, cnt0_hbm, cnt1_hbm,
         agg_sh, rows0_v, rows1_v, isrc_v, idst_v,
         si0, si1, sg0, sg1, ss0, ss1, sc0, sc1,
         cnt_sh, ones_v, zc_v) = refs
        scnt = (sc0, sc1)
    else:
        (agg0_hbm, agg1_hbm, agg_sh, rows0_v, rows1_v, isrc_v, idst_v,
         si0, si1, sg0, sg1, ss0, ss1) = refs
        scnt = (None, None)
    rows = (rows0_v, rows1_v)
    sidx = (si0, si1)
    sgat = (sg0, sg1)
    ssct = (ss0, ss1)

    cid = lax.axis_index("c")
    sid = lax.axis_index("s")
    wid = cid * NS + sid

    @pl.loop(0, CHUNK)
    def _(r):
        @pl.loop(0, D, step=16)
        def _(j):
            rows0_v[r, pl.ds(j, 16)] = jnp.zeros((16,), jnp.float32)

    start = sid * ROW_STRIDE

    @pl.loop(0, 5)
    def _(k):
        pltpu.sync_copy(rows0_v, agg_sh.at[pl.ds(start + k * CHUNK, CHUNK)])

    if with_cnt:
        @pl.loop(0, CHUNK, step=16)
        def _(j):
            ones_v[pl.ds(j, 16)] = jnp.ones((16,), jnp.float32)

        @pl.when(sid == 0)
        def _():
            @pl.loop(0, N, step=16)
            def _(j):
                zc_v[pl.ds(j, 16)] = jnp.zeros((16,), jnp.float32)
            pltpu.sync_copy(zc_v, cnt_sh)

    plsc.subcore_barrier()

    def idx_start(i, b):
        base = (i * NT + wid) * CHUNK
        pltpu.async_copy(src_hbm.at[pl.ds(base, CHUNK)], isrc_v.at[b], sidx[b])
        pltpu.async_copy(dst_hbm.at[pl.ds(base, CHUNK)], idst_v.at[b], sidx[b])

    def idx_wait(b):
        pltpu.make_async_copy(src_hbm.at[pl.ds(0, CHUNK)], isrc_v.at[b],
                              sidx[b]).wait()
        pltpu.make_async_copy(dst_hbm.at[pl.ds(0, CHUNK)], idst_v.at[b],
                              sidx[b]).wait()

    def gather_start(b):
        pltpu.async_copy(x_hbm.at[isrc_v.at[b]], rows[b], sgat[b])

    def gather_wait(b):
        pltpu.make_async_copy(x_hbm.at[isrc_v.at[b]], rows[b], sgat[b]).wait()

    def scat_start(b):
        pltpu.async_copy(rows[b], agg_sh.at[idst_v.at[b]], ssct[b], add=True)
        if with_cnt:
            pltpu.async_copy(ones_v, cnt_sh.at[idst_v.at[b]], scnt[b],
                             add=True)

    def scat_wait(b):
        pltpu.make_async_copy(rows[b], agg_sh.at[idst_v.at[b]],
                              ssct[b]).wait()
        if with_cnt:
            pltpu.make_async_copy(ones_v, cnt_sh.at[idst_v.at[b]],
                                  scnt[b]).wait()

    idx_start(0, 0)

    @pl.loop(0, 39)
    def _(j):
        i0 = 2 * j
        idx_wait(0)
        gather_start(0)

        @pl.when(j > 0)
        def _():
            scat_wait(1)

        idx_start(i0 + 1, 1)
        gather_wait(0)
        scat_start(0)
        idx_wait(1)
        gather_start(1)
        scat_wait(0)

        @pl.when(j < 38)
        def _():
            idx_start(i0 + 2, 0)

        gather_wait(1)
        scat_start(1)

    scat_wait(1)

    @pl.when(wid < 4)
    def _():
        base = (78 * NT + wid) * CHUNK
        pltpu.sync_copy(src_hbm.at[pl.ds(base, CHUNK)], isrc_v.at[0])
        pltpu.sync_copy(dst_hbm.at[pl.ds(base, CHUNK)], idst_v.at[0])
        pltpu.sync_copy(x_hbm.at[isrc_v.at[0]], rows0_v)
        pltpu.sync_copy(rows0_v, agg_sh.at[idst_v.at[0]], add=True)
        if with_cnt:
            pltpu.sync_copy(ones_v, cnt_sh.at[idst_v.at[0]], add=True)

    plsc.subcore_barrier()

    def _copy_out(dst_hbm_ref):
        pltpu.sync_copy(agg_sh.at[pl.ds(start, 512)],
                        dst_hbm_ref.at[pl.ds(start, 512)])

        @pl.when(sid < NS - 1)
        def _():
            pltpu.sync_copy(agg_sh.at[pl.ds(start + 512, 112)],
                            dst_hbm_ref.at[pl.ds(start + 512, 112)])

        @pl.when(sid == NS - 1)
        def _():
            pltpu.sync_copy(agg_sh.at[pl.ds(start + 512, 128)],
                            dst_hbm_ref.at[pl.ds(start + 512, 128)])

    @pl.when(cid == 0)
    def _():
        _copy_out(agg0_hbm)

    @pl.when(cid == 1)
    def _():
        _copy_out(agg1_hbm)

    if with_cnt:
        @pl.when((sid == 0) & (cid == 0))
        def _():
            pltpu.sync_copy(cnt_sh, cnt0_hbm)

        @pl.when((sid == 0) & (cid == 1))
        def _():
            pltpu.sync_copy(cnt_sh, cnt1_hbm)


def _make_sc_agg(with_cnt):
    mesh = plsc.VectorSubcoreMesh(core_axis_name="c", subcore_axis_name="s",
                                  num_cores=NC, num_subcores=NS)
    out_type = [jax.ShapeDtypeStruct((N, D), jnp.float32),
                jax.ShapeDtypeStruct((N, D), jnp.float32)]
    scratch = [
        pltpu.VMEM_SHARED((N, D), jnp.float32),
        pltpu.VMEM((CHUNK, D), jnp.float32),
        pltpu.VMEM((CHUNK, D), jnp.float32),
        pltpu.VMEM((2, CHUNK), jnp.int32),
        pltpu.VMEM((2, CHUNK), jnp.int32),
        pltpu.SemaphoreType.DMA,
        pltpu.SemaphoreType.DMA,
        pltpu.SemaphoreType.DMA,
        pltpu.SemaphoreType.DMA,
        pltpu.SemaphoreType.DMA,
        pltpu.SemaphoreType.DMA,
    ]
    if with_cnt:
        out_type += [jax.ShapeDtypeStruct((N,), jnp.float32),
                     jax.ShapeDtypeStruct((N,), jnp.float32)]
        scratch += [
            pltpu.SemaphoreType.DMA,
            pltpu.SemaphoreType.DMA,
            pltpu.VMEM_SHARED((N,), jnp.float32),
            pltpu.VMEM((CHUNK,), jnp.float32),
            pltpu.VMEM((N,), jnp.float32),
        ]
    return pl.kernel(functools.partial(_sc_agg_body, with_cnt),
                     out_type=tuple(out_type), mesh=mesh,
                     scratch_types=scratch)


_sc_agg_cnt = _make_sc_agg(True)
_sc_agg = _make_sc_agg(False)


_BQ = 400
_DN = (((1,), (1,)), ((), ()))


def _dense_body(relu, a0_ref, a1_ref, c0_ref, c1_ref, x_ref, wl_ref, bl_ref,
                wr_ref, o_ref):
    cnt = c0_ref[...] + c1_ref[...]
    inv = 1.0 / jnp.maximum(cnt, 1.0)
    mean = (a0_ref[...] + a1_ref[...]) * inv
    acc = lax.dot_general(mean, wl_ref[...], _DN,
                          preferred_element_type=jnp.float32,
                          precision=lax.Precision.HIGHEST)
    acc = acc + lax.dot_general(x_ref[...], wr_ref[...], _DN,
                                preferred_element_type=jnp.float32,
                                precision=lax.Precision.HIGHEST)
    acc = acc + bl_ref[...]
    if relu:
        acc = jnp.maximum(acc, 0.0)
    o_ref[...] = acc


def _dense(relu, a0, a1, c0, c1, x, wl, bl, wr):
    row_spec = pl.BlockSpec((_BQ, D), lambda i: (i, 0))
    cnt_spec = pl.BlockSpec((_BQ, 1), lambda i: (i, 0))
    w_spec = pl.BlockSpec((D, D), lambda i: (0, 0))
    b_spec = pl.BlockSpec((1, D), lambda i: (0, 0))
    return pl.pallas_call(
        functools.partial(_dense_body, relu),
        grid=(N // _BQ,),
        in_specs=[row_spec, row_spec, cnt_spec, cnt_spec, row_spec,
                  w_spec, b_spec, w_spec],
        out_specs=row_spec,
        out_shape=jax.ShapeDtypeStruct((N, D), jnp.float32),
    )(a0, a1, c0, c1, x, wl, bl, wr)


def kernel(x, edge_index, W1l, b1l, W1r, W2l, b2l, W2r):
    src = edge_index[0]
    dst = edge_index[1]
    agg0, agg1, cnt0, cnt1 = _sc_agg_cnt(x, src, dst)
    c0 = cnt0.reshape(N, 1)
    c1 = cnt1.reshape(N, 1)
    h = _dense(True, agg0, agg1, c0, c1, x, W1l, b1l.reshape(1, D), W1r)
    b0, b1_ = _sc_agg(h, src, dst)
    out = _dense(False, b0, b1_, c0, c1, h, W2l, b2l.reshape(1, D), W2r)
    return out


# speedup vs baseline: 11.0778x; 1.0157x over previous
import functools

import jax
import jax.numpy as jnp
from jax import lax
from jax.experimental import pallas as pl
from jax.experimental.pallas import tpu as pltpu
from jax.experimental.pallas import tpu_sc as plsc

N = 10000
E = 320000
D = 128

NC = 2
NS = 16
NT = NC * NS

CHUNK = 128
NCHUNKS = E // CHUNK
CPT = (NCHUNKS + NT - 1) // NT
ROW_STRIDE = 624


NBUF = 3


def _sc_agg_body(with_cnt, *refs):
    if with_cnt:
        (x_hbm, src_hbm, dst_hbm, zeros_hbm,
         agg0_hbm, agg1_hbm, cnt0_hbm, cnt1_hbm) = refs[:8]
        refs = refs[8:]
    else:
        x_hbm, src_hbm, dst_hbm, agg0_hbm, agg1_hbm = refs[:5]
        refs = refs[5:]
    agg_sh = refs[0]
    rows = refs[1:1 + NBUF]
    isrc_v, idst_v = refs[1 + NBUF:3 + NBUF]
    k = 3 + NBUF
    sidx = refs[k:k + NBUF]
    sgat = refs[k + NBUF:k + 2 * NBUF]
    ssct = refs[k + 2 * NBUF:k + 3 * NBUF]
    k += 3 * NBUF
    if with_cnt:
        scnt = refs[k:k + NBUF]
        cnt_sh, ones_v = refs[k + NBUF:]
    rows0_v = rows[0]

    cid = lax.axis_index("c")
    sid = lax.axis_index("s")
    wid = cid * NS + sid

    @pl.loop(0, CHUNK)
    def _(r):
        @pl.loop(0, D, step=16)
        def _(j):
            rows0_v[r, pl.ds(j, 16)] = jnp.zeros((16,), jnp.float32)

    start = sid * ROW_STRIDE

    @pl.loop(0, 5)
    def _(k):
        pltpu.sync_copy(rows0_v, agg_sh.at[pl.ds(start + k * CHUNK, CHUNK)])

    if with_cnt:
        @pl.loop(0, CHUNK, step=16)
        def _(j):
            ones_v[pl.ds(j, 16)] = jnp.ones((16,), jnp.float32)

        @pl.when(sid == 0)
        def _():
            pltpu.sync_copy(zeros_hbm, cnt_sh)

    plsc.subcore_barrier()

    def idx_start(i, b):
        base = (i * NT + wid) * CHUNK
        pltpu.async_copy(src_hbm.at[pl.ds(base, CHUNK)], isrc_v.at[b], sidx[b])
        pltpu.async_copy(dst_hbm.at[pl.ds(base, CHUNK)], idst_v.at[b], sidx[b])

    def idx_wait(b):
        pltpu.make_async_copy(src_hbm.at[pl.ds(0, CHUNK)], isrc_v.at[b],
                              sidx[b]).wait()
        pltpu.make_async_copy(dst_hbm.at[pl.ds(0, CHUNK)], idst_v.at[b],
                              sidx[b]).wait()

    def gather_start(b):
        pltpu.async_copy(x_hbm.at[isrc_v.at[b]], rows[b], sgat[b])

    def gather_wait(b):
        pltpu.make_async_copy(x_hbm.at[isrc_v.at[b]], rows[b], sgat[b]).wait()

    def scat_start(b):
        pltpu.async_copy(rows[b], agg_sh.at[idst_v.at[b]], ssct[b], add=True)
        if with_cnt:
            pltpu.async_copy(ones_v, cnt_sh.at[idst_v.at[b]], scnt[b],
                             add=True)

    def scat_wait(b):
        pltpu.make_async_copy(rows[b], agg_sh.at[idst_v.at[b]],
                              ssct[b]).wait()
        if with_cnt:
            pltpu.make_async_copy(ones_v, cnt_sh.at[idst_v.at[b]],
                                  scnt[b]).wait()

    idx_start(0, 0)
    idx_start(1, 1)
    idx_wait(0)
    gather_start(0)

    @pl.loop(0, 26)
    def _(j):
        for b in range(NBUF):
            i = 3 * j + b
            gather_wait(b)
            scat_start(b)
            prv = (b + 2) % NBUF
            if b == 0:
                @pl.when(j > 0)
                def _():
                    scat_wait(prv)

                idx_start(i + 2, prv)
            else:
                scat_wait(prv)
                if b == 1:
                    @pl.when((j < 25) | (wid < 4))
                    def _():
                        idx_start(i + 2, prv)
                else:
                    @pl.when(j < 25)
                    def _():
                        idx_start(i + 2, prv)
            nb = (b + 1) % NBUF
            if b < 2:
                idx_wait(nb)
                gather_start(nb)
            else:
                @pl.when((j < 25) | (wid < 4))
                def _():
                    idx_wait(nb)
                    gather_start(nb)

    @pl.when(wid < 4)
    def _():
        gather_wait(0)
        scat_start(0)

    scat_wait(2)

    @pl.when(wid < 4)
    def _():
        scat_wait(0)

    plsc.subcore_barrier()

    def _copy_out(dst_hbm_ref):
        pltpu.sync_copy(agg_sh.at[pl.ds(start, 512)],
                        dst_hbm_ref.at[pl.ds(start, 512)])

        @pl.when(sid < NS - 1)
        def _():
            pltpu.sync_copy(agg_sh.at[pl.ds(start + 512, 112)],
                            dst_hbm_ref.at[pl.ds(start + 512, 112)])

        @pl.when(sid == NS - 1)
        def _():
            pltpu.sync_copy(agg_sh.at[pl.ds(start + 512, 128)],
                            dst_hbm_ref.at[pl.ds(start + 512, 128)])

    @pl.when(cid == 0)
    def _():
        _copy_out(agg0_hbm)

    @pl.when(cid == 1)
    def _():
        _copy_out(agg1_hbm)

    if with_cnt:
        @pl.when((sid == 0) & (cid == 0))
        def _():
            pltpu.sync_copy(cnt_sh, cnt0_hbm)

        @pl.when((sid == 0) & (cid == 1))
        def _():
            pltpu.sync_copy(cnt_sh, cnt1_hbm)


def _make_sc_agg(with_cnt):
    mesh = plsc.VectorSubcoreMesh(core_axis_name="c", subcore_axis_name="s",
                                  num_cores=NC, num_subcores=NS)
    out_type = [jax.ShapeDtypeStruct((N, D), jnp.float32),
                jax.ShapeDtypeStruct((N, D), jnp.float32)]
    scratch = (
        [pltpu.VMEM_SHARED((N, D), jnp.float32)]
        + [pltpu.VMEM((CHUNK, D), jnp.float32)] * NBUF
        + [pltpu.VMEM((NBUF, CHUNK), jnp.int32)] * 2
        + [pltpu.SemaphoreType.DMA] * (3 * NBUF)
    )
    if with_cnt:
        out_type += [jax.ShapeDtypeStruct((N,), jnp.float32),
                     jax.ShapeDtypeStruct((N,), jnp.float32)]
        scratch += (
            [pltpu.SemaphoreType.DMA] * NBUF +
            [pltpu.VMEM_SHARED((N,), jnp.float32),
             pltpu.VMEM((CHUNK,), jnp.float32)]
        )
    return pl.kernel(functools.partial(_sc_agg_body, with_cnt),
                     out_type=tuple(out_type), mesh=mesh,
                     scratch_types=scratch)


_sc_agg_cnt = _make_sc_agg(True)
_sc_agg = _make_sc_agg(False)


_BQ = 400
_DN = (((1,), (1,)), ((), ()))


def _dense_body(relu, a0_ref, a1_ref, c0_ref, c1_ref, x_ref, wl_ref, bl_ref,
                wr_ref, o_ref):
    cnt = c0_ref[...] + c1_ref[...]
    inv = 1.0 / jnp.maximum(cnt, 1.0)
    mean = (a0_ref[...] + a1_ref[...]) * inv
    acc = lax.dot_general(mean, wl_ref[...], _DN,
                          preferred_element_type=jnp.float32,
                          precision=lax.Precision.HIGHEST)
    acc = acc + lax.dot_general(x_ref[...], wr_ref[...], _DN,
                                preferred_element_type=jnp.float32,
                                precision=lax.Precision.HIGHEST)
    acc = acc + bl_ref[...]
    if relu:
        acc = jnp.maximum(acc, 0.0)
    o_ref[...] = acc


def _dense(relu, a0, a1, c0, c1, x, wl, bl, wr):
    row_spec = pl.BlockSpec((_BQ, D), lambda i: (i, 0))
    cnt_spec = pl.BlockSpec((_BQ, 1), lambda i: (i, 0))
    w_spec = pl.BlockSpec((D, D), lambda i: (0, 0))
    b_spec = pl.BlockSpec((1, D), lambda i: (0, 0))
    return pl.pallas_call(
        functools.partial(_dense_body, relu),
        grid=(N // _BQ,),
        in_specs=[row_spec, row_spec, cnt_spec, cnt_spec, row_spec,
                  w_spec, b_spec, w_spec],
        out_specs=row_spec,
        out_shape=jax.ShapeDtypeStruct((N, D), jnp.float32),
    )(a0, a1, c0, c1, x, wl, bl, wr)


def kernel(x, edge_index, W1l, b1l, W1r, W2l, b2l, W2r):
    src = edge_index[0]
    dst = edge_index[1]
    zeros = jnp.zeros((N,), jnp.float32)
    agg0, agg1, cnt0, cnt1 = _sc_agg_cnt(x, src, dst, zeros)
    c0 = cnt0.reshape(N, 1)
    c1 = cnt1.reshape(N, 1)
    h = _dense(True, agg0, agg1, c0, c1, x, W1l, b1l.reshape(1, D), W1r)
    b0, b1_ = _sc_agg(h, src, dst)
    out = _dense(False, b0, b1_, c0, c1, h, W2l, b2l.reshape(1, D), W2r)
    return out


# speedup vs baseline: 13.2113x; 1.1926x over previous
import functools

import jax
import jax.numpy as jnp
from jax import lax
from jax.experimental import pallas as pl
from jax.experimental.pallas import tpu as pltpu
from jax.experimental.pallas import tpu_sc as plsc

N = 10000
E = 320000
D = 128

NC = 2
NS = 16
NT = NC * NS

CHUNK = 128
NCHUNKS = E // CHUNK
CPT = (NCHUNKS + NT - 1) // NT
ROW_STRIDE = 624


NBUF = 3


def _sc_agg_body(with_cnt, *refs):
    if with_cnt:
        (x_hbm, e3_hbm, zeros_hbm,
         agg0_hbm, agg1_hbm, cnt0_hbm, cnt1_hbm) = refs[:7]
        refs = refs[7:]
    else:
        x_hbm, e3_hbm, agg0_hbm, agg1_hbm = refs[:4]
        refs = refs[4:]
    agg_sh = refs[0]
    rows = refs[1:1 + NBUF]
    idx_v = refs[1 + NBUF]
    k = 2 + NBUF
    sidx = refs[k:k + NBUF]
    sgat = refs[k + NBUF:k + 2 * NBUF]
    ssct = refs[k + 2 * NBUF:k + 3 * NBUF]
    k += 3 * NBUF
    if with_cnt:
        scnt = refs[k:k + NBUF]
        cnt_sh, ones_v = refs[k + NBUF:]
    rows0_v = rows[0]

    cid = lax.axis_index("c")
    sid = lax.axis_index("s")
    wid = cid * NS + sid

    @pl.loop(0, CHUNK)
    def _(r):
        @pl.loop(0, D, step=16)
        def _(j):
            rows0_v[r, pl.ds(j, 16)] = jnp.zeros((16,), jnp.float32)

    start = sid * ROW_STRIDE

    @pl.loop(0, 5)
    def _(k):
        pltpu.sync_copy(rows0_v, agg_sh.at[pl.ds(start + k * CHUNK, CHUNK)])

    if with_cnt:
        @pl.loop(0, CHUNK, step=16)
        def _(j):
            ones_v[pl.ds(j, 16)] = jnp.ones((16,), jnp.float32)

        @pl.when(sid == 0)
        def _():
            pltpu.sync_copy(zeros_hbm, cnt_sh)

    plsc.subcore_barrier()

    def idx_start(i, b):
        pltpu.async_copy(e3_hbm.at[i * NT + wid], idx_v.at[b], sidx[b])

    def idx_wait(b):
        pltpu.make_async_copy(e3_hbm.at[0], idx_v.at[b], sidx[b]).wait()

    def gather_start(b):
        pltpu.async_copy(x_hbm.at[idx_v.at[b, 0]], rows[b], sgat[b])

    def gather_wait(b):
        pltpu.make_async_copy(x_hbm.at[idx_v.at[b, 0]], rows[b],
                              sgat[b]).wait()

    def scat_start(b):
        pltpu.async_copy(rows[b], agg_sh.at[idx_v.at[b, 1]], ssct[b],
                         add=True)
        if with_cnt:
            pltpu.async_copy(ones_v, cnt_sh.at[idx_v.at[b, 1]], scnt[b],
                             add=True)

    def scat_wait(b):
        pltpu.make_async_copy(rows[b], agg_sh.at[idx_v.at[b, 1]],
                              ssct[b]).wait()
        if with_cnt:
            pltpu.make_async_copy(ones_v, cnt_sh.at[idx_v.at[b, 1]],
                                  scnt[b]).wait()

    idx_start(0, 0)
    idx_start(1, 1)
    idx_wait(0)
    gather_start(0)

    @pl.loop(0, 26)
    def _(j):
        for b in range(NBUF):
            i = 3 * j + b
            nb = (b + 1) % NBUF
            prv = (b + 2) % NBUF

            if b == 2:
                @pl.when((j < 25) | (wid < 4))
                def _():
                    idx_wait(nb)
                    gather_start(nb)
            else:
                idx_wait(nb)
                gather_start(nb)
            gather_wait(b)
            scat_start(b)
            if b == 0:
                @pl.when(j > 0)
                def _():
                    scat_wait(prv)
            else:
                scat_wait(prv)
            if b == 0:
                idx_start(i + 2, prv)
            elif b == 1:
                @pl.when((j < 25) | (wid < 4))
                def _():
                    idx_start(i + 2, prv)
            else:
                @pl.when(j < 25)
                def _():
                    idx_start(i + 2, prv)

    @pl.when(wid < 4)
    def _():
        gather_wait(0)
        scat_start(0)

    scat_wait(2)

    @pl.when(wid < 4)
    def _():
        scat_wait(0)

    plsc.subcore_barrier()

    def _copy_out(dst_hbm_ref):
        pltpu.sync_copy(agg_sh.at[pl.ds(start, 512)],
                        dst_hbm_ref.at[pl.ds(start, 512)])

        @pl.when(sid < NS - 1)
        def _():
            pltpu.sync_copy(agg_sh.at[pl.ds(start + 512, 112)],
                            dst_hbm_ref.at[pl.ds(start + 512, 112)])

        @pl.when(sid == NS - 1)
        def _():
            pltpu.sync_copy(agg_sh.at[pl.ds(start + 512, 128)],
                            dst_hbm_ref.at[pl.ds(start + 512, 128)])

    @pl.when(cid == 0)
    def _():
        _copy_out(agg0_hbm)

    @pl.when(cid == 1)
    def _():
        _copy_out(agg1_hbm)

    if with_cnt:
        @pl.when((sid == 0) & (cid == 0))
        def _():
            pltpu.sync_copy(cnt_sh, cnt0_hbm)

        @pl.when((sid == 0) & (cid == 1))
        def _():
            pltpu.sync_copy(cnt_sh, cnt1_hbm)


def _make_sc_agg(with_cnt):
    mesh = plsc.VectorSubcoreMesh(core_axis_name="c", subcore_axis_name="s",
                                  num_cores=NC, num_subcores=NS)
    out_type = [jax.ShapeDtypeStruct((N, D), jnp.float32),
                jax.ShapeDtypeStruct((N, D), jnp.float32)]
    scratch = (
        [pltpu.VMEM_SHARED((N, D), jnp.float32)]
        + [pltpu.VMEM((CHUNK, D), jnp.float32)] * NBUF
        + [pltpu.VMEM((NBUF, 2, CHUNK), jnp.int32)]
        + [pltpu.SemaphoreType.DMA] * (3 * NBUF)
    )
    if with_cnt:
        out_type += [jax.ShapeDtypeStruct((N,), jnp.float32),
                     jax.ShapeDtypeStruct((N,), jnp.float32)]
        scratch += (
            [pltpu.SemaphoreType.DMA] * NBUF +
            [pltpu.VMEM_SHARED((N,), jnp.float32),
             pltpu.VMEM((CHUNK,), jnp.float32)]
        )
    return pl.kernel(functools.partial(_sc_agg_body, with_cnt),
                     out_type=tuple(out_type), mesh=mesh,
                     scratch_types=scratch)


_sc_agg_cnt = _make_sc_agg(True)
_sc_agg = _make_sc_agg(False)


_BQ = 400
_DN = (((1,), (1,)), ((), ()))


def _dense_body(relu, a0_ref, a1_ref, c0_ref, c1_ref, x_ref, wl_ref, bl_ref,
                wr_ref, o_ref):
    cnt = c0_ref[...] + c1_ref[...]
    inv = 1.0 / jnp.maximum(cnt, 1.0)
    mean = (a0_ref[...] + a1_ref[...]) * inv
    acc = lax.dot_general(mean, wl_ref[...], _DN,
                          preferred_element_type=jnp.float32,
                          precision=lax.Precision.HIGHEST)
    acc = acc + lax.dot_general(x_ref[...], wr_ref[...], _DN,
                                preferred_element_type=jnp.float32,
                                precision=lax.Precision.HIGHEST)
    acc = acc + bl_ref[...]
    if relu:
        acc = jnp.maximum(acc, 0.0)
    o_ref[...] = acc


def _dense(relu, a0, a1, c0, c1, x, wl, bl, wr):
    row_spec = pl.BlockSpec((_BQ, D), lambda i: (i, 0))
    cnt_spec = pl.BlockSpec((_BQ, 1), lambda i: (i, 0))
    w_spec = pl.BlockSpec((D, D), lambda i: (0, 0))
    b_spec = pl.BlockSpec((1, D), lambda i: (0, 0))
    return pl.pallas_call(
        functools.partial(_dense_body, relu),
        grid=(N // _BQ,),
        in_specs=[row_spec, row_spec, cnt_spec, cnt_spec, row_spec,
                  w_spec, b_spec, w_spec],
        out_specs=row_spec,
        out_shape=jax.ShapeDtypeStruct((N, D), jnp.float32),
    )(a0, a1, c0, c1, x, wl, bl, wr)


def kernel(x, edge_index, W1l, b1l, W1r, W2l, b2l, W2r):
    e3 = edge_index.reshape(2, NCHUNKS, CHUNK).transpose(1, 0, 2)
    zeros = jnp.zeros((N,), jnp.float32)
    agg0, agg1, cnt0, cnt1 = _sc_agg_cnt(x, e3, zeros)
    c0 = cnt0.reshape(N, 1)
    c1 = cnt1.reshape(N, 1)
    h = _dense(True, agg0, agg1, c0, c1, x, W1l, b1l.reshape(1, D), W1r)
    b0, b1_ = _sc_agg(h, e3)
    out = _dense(False, b0, b1_, c0, c1, h, W2l, b2l.reshape(1, D), W2r)
    return out


# speedup vs baseline: 13.3635x; 1.0115x over previous
import functools

import jax
import jax.numpy as jnp
from jax import lax
from jax.experimental import pallas as pl
from jax.experimental.pallas import tpu as pltpu
from jax.experimental.pallas import tpu_sc as plsc

N = 10000
E = 320000
D = 128

NC = 2
NS = 16
NT = NC * NS

CHUNK = 128
NCHUNKS = E // CHUNK
CPT = (NCHUNKS + NT - 1) // NT
ROW_STRIDE = 624


NBUF = 3


def _sc_agg_body(with_cnt, *refs):
    if with_cnt:
        (x_hbm, e3_hbm, zeros_hbm,
         agg0_hbm, agg1_hbm, cnt0_hbm, cnt1_hbm) = refs[:7]
        refs = refs[7:]
    else:
        x_hbm, e3_hbm, agg0_hbm, agg1_hbm = refs[:4]
        refs = refs[4:]
    agg_sh = refs[0]
    rows = refs[1:1 + NBUF]
    idx_v = refs[1 + NBUF]
    k = 2 + NBUF
    sidx = refs[k:k + NBUF]
    sgat = refs[k + NBUF:k + 2 * NBUF]
    ssct = refs[k + 2 * NBUF:k + 3 * NBUF]
    k += 3 * NBUF
    if with_cnt:
        scnt = refs[k:k + NBUF]
        cnt_sh, ones_v = refs[k + NBUF:]
    rows0_v = rows[0]

    cid = lax.axis_index("c")
    sid = lax.axis_index("s")
    wid = cid * NS + sid

    @pl.loop(0, CHUNK)
    def _(r):
        @pl.loop(0, D, step=16)
        def _(j):
            rows0_v[r, pl.ds(j, 16)] = jnp.zeros((16,), jnp.float32)

    start = sid * ROW_STRIDE

    @pl.loop(0, 5)
    def _(k):
        pltpu.sync_copy(rows0_v, agg_sh.at[pl.ds(start + k * CHUNK, CHUNK)])

    if with_cnt:
        @pl.loop(0, CHUNK, step=16)
        def _(j):
            ones_v[pl.ds(j, 16)] = jnp.ones((16,), jnp.float32)

        @pl.when(sid == 0)
        def _():
            pltpu.sync_copy(zeros_hbm, cnt_sh)

    plsc.subcore_barrier()

    def idx_start(i, b):
        pltpu.async_copy(e3_hbm.at[i * NT + wid], idx_v.at[b], sidx[b])

    def idx_wait(b):
        pltpu.make_async_copy(e3_hbm.at[0], idx_v.at[b], sidx[b]).wait()

    def gather_start(b):
        pltpu.async_copy(x_hbm.at[idx_v.at[b, 0]], rows[b], sgat[b])

    def gather_wait(b):
        pltpu.make_async_copy(x_hbm.at[idx_v.at[b, 0]], rows[b],
                              sgat[b]).wait()

    def scat_start(b):
        pltpu.async_copy(rows[b], agg_sh.at[idx_v.at[b, 1]], ssct[b],
                         add=True)
        if with_cnt:
            pltpu.async_copy(ones_v, cnt_sh.at[idx_v.at[b, 1]], scnt[b],
                             add=True)

    def scat_wait(b):
        pltpu.make_async_copy(rows[b], agg_sh.at[idx_v.at[b, 1]],
                              ssct[b]).wait()
        if with_cnt:
            pltpu.make_async_copy(ones_v, cnt_sh.at[idx_v.at[b, 1]],
                                  scnt[b]).wait()

    idx_start(0, 0)
    idx_start(1, 1)
    idx_wait(0)
    gather_start(0)

    @pl.loop(0, 26)
    def _(j):
        for b in range(NBUF):
            i = 3 * j + b
            nb = (b + 1) % NBUF
            prv = (b + 2) % NBUF

            if b == 2:
                @pl.when((j < 25) | (wid < 4))
                def _():
                    idx_wait(nb)
                    gather_start(nb)
            else:
                idx_wait(nb)
                gather_start(nb)
            gather_wait(b)
            scat_start(b)
            if b == 0:
                @pl.when(j > 0)
                def _():
                    scat_wait(prv)
            else:
                scat_wait(prv)
            if b == 0:
                idx_start(i + 2, prv)
            elif b == 1:
                @pl.when((j < 25) | (wid < 4))
                def _():
                    idx_start(i + 2, prv)
            else:
                @pl.when(j < 25)
                def _():
                    idx_start(i + 2, prv)

    @pl.when(wid < 4)
    def _():
        gather_wait(0)
        scat_start(0)

    scat_wait(2)

    @pl.when(wid < 4)
    def _():
        scat_wait(0)

    plsc.subcore_barrier()

    def _copy_out(dst_hbm_ref):
        pltpu.sync_copy(agg_sh.at[pl.ds(start, 512)],
                        dst_hbm_ref.at[pl.ds(start, 512)])

        @pl.when(sid < NS - 1)
        def _():
            pltpu.sync_copy(agg_sh.at[pl.ds(start + 512, 112)],
                            dst_hbm_ref.at[pl.ds(start + 512, 112)])

        @pl.when(sid == NS - 1)
        def _():
            pltpu.sync_copy(agg_sh.at[pl.ds(start + 512, 128)],
                            dst_hbm_ref.at[pl.ds(start + 512, 128)])

    @pl.when(cid == 0)
    def _():
        _copy_out(agg0_hbm)

    @pl.when(cid == 1)
    def _():
        _copy_out(agg1_hbm)

    if with_cnt:
        @pl.when((sid == 0) & (cid == 0))
        def _():
            pltpu.sync_copy(cnt_sh, cnt0_hbm)

        @pl.when((sid == 0) & (cid == 1))
        def _():
            pltpu.sync_copy(cnt_sh, cnt1_hbm)


def _make_sc_agg(with_cnt):
    mesh = plsc.VectorSubcoreMesh(core_axis_name="c", subcore_axis_name="s",
                                  num_cores=NC, num_subcores=NS)
    out_type = [jax.ShapeDtypeStruct((N, D), jnp.float32),
                jax.ShapeDtypeStruct((N, D), jnp.float32)]
    scratch = (
        [pltpu.VMEM_SHARED((N, D), jnp.float32)]
        + [pltpu.VMEM((CHUNK, D), jnp.float32)] * NBUF
        + [pltpu.VMEM((NBUF, 2, CHUNK), jnp.int32)]
        + [pltpu.SemaphoreType.DMA] * (3 * NBUF)
    )
    if with_cnt:
        out_type += [jax.ShapeDtypeStruct((N,), jnp.float32),
                     jax.ShapeDtypeStruct((N,), jnp.float32)]
        scratch += (
            [pltpu.SemaphoreType.DMA] * NBUF +
            [pltpu.VMEM_SHARED((N,), jnp.float32),
             pltpu.VMEM((CHUNK,), jnp.float32)]
        )
    return pl.kernel(functools.partial(_sc_agg_body, with_cnt),
                     out_type=tuple(out_type), mesh=mesh,
                     scratch_types=scratch)


_sc_agg_cnt = _make_sc_agg(True)
_sc_agg = _make_sc_agg(False)


_BQ = 400
_DN = (((1,), (1,)), ((), ()))


_ROW_SPEC = pl.BlockSpec((_BQ, D), lambda i: (i, 0))
_CNT_SPEC = pl.BlockSpec((_BQ, 1), lambda i: (i, 0))
_W_SPEC = pl.BlockSpec((D, D), lambda i: (0, 0))
_B_SPEC = pl.BlockSpec((1, D), lambda i: (0, 0))
_OUT_ND = jax.ShapeDtypeStruct((N, D), jnp.float32)


def _dense_r_body(x_ref, wr_ref, b_ref, o_ref):
    o_ref[...] = lax.dot_general(x_ref[...], wr_ref[...], _DN,
                                 preferred_element_type=jnp.float32,
                                 precision=lax.Precision.HIGHEST) + b_ref[...]


def _dense_r(x, wr, b):
    return pl.pallas_call(
        _dense_r_body,
        grid=(N // _BQ,),
        in_specs=[_ROW_SPEC, _W_SPEC, _B_SPEC],
        out_specs=_ROW_SPEC,
        out_shape=_OUT_ND,
    )(x, wr, b)


def _dense_l_body(relu, a0_ref, a1_ref, c0_ref, c1_ref, xr_ref, wl_ref,
                  o_ref):
    cnt = c0_ref[...] + c1_ref[...]
    inv = 1.0 / jnp.maximum(cnt, 1.0)
    mean = (a0_ref[...] + a1_ref[...]) * inv
    acc = lax.dot_general(mean, wl_ref[...], _DN,
                          preferred_element_type=jnp.float32,
                          precision=lax.Precision.HIGHEST)
    acc = acc + xr_ref[...]
    if relu:
        acc = jnp.maximum(acc, 0.0)
    o_ref[...] = acc


def _dense_l(relu, a0, a1, c0, c1, xr, wl):
    return pl.pallas_call(
        functools.partial(_dense_l_body, relu),
        grid=(N // _BQ,),
        in_specs=[_ROW_SPEC, _ROW_SPEC, _CNT_SPEC, _CNT_SPEC, _ROW_SPEC,
                  _W_SPEC],
        out_specs=_ROW_SPEC,
        out_shape=_OUT_ND,
    )(a0, a1, c0, c1, xr, wl)


def kernel(x, edge_index, W1l, b1l, W1r, W2l, b2l, W2r):
    e3 = edge_index.reshape(2, NCHUNKS, CHUNK).transpose(1, 0, 2)
    zeros = jnp.zeros((N,), jnp.float32)
    xr1 = _dense_r(x, W1r, b1l.reshape(1, D))
    agg0, agg1, cnt0, cnt1 = _sc_agg_cnt(x, e3, zeros)
    c0 = cnt0.reshape(N, 1)
    c1 = cnt1.reshape(N, 1)
    h = _dense_l(True, agg0, agg1, c0, c1, xr1, W1l)
    hr = _dense_r(h, W2r, b2l.reshape(1, D))
    b0, b1_ = _sc_agg(h, e3)
    out = _dense_l(False, b0, b1_, c0, c1, hr, W2l)
    return out
